# baseline (speedup 1.0000x reference)
import functools

import jax
import jax.numpy as jnp
from jax import lax
from jax.experimental import pallas as pl
from jax.experimental.pallas import tpu as pltpu
from jax.experimental.pallas import tpu_sc as plsc

R, S = 4096, 200
D = 32
V = 1000000
NC, NS = 2, 16
NW = NC * NS
SB, S8 = S // 8, 8
CB, CL = R // 128, 128
D8, DL = D // 8, 8

_mesh = plsc.VectorSubcoreMesh(core_axis_name="c", subcore_axis_name="s")


@functools.partial(
    pl.kernel,
    out_type=jax.ShapeDtypeStruct((S, D8, CB, DL, CL), jnp.float32),
    mesh=_mesh,
    scratch_types=[
        pltpu.VMEM((SB, S8, CL), jnp.int32),
        pltpu.VMEM((CL, D), jnp.float32),
        pltpu.VMEM((CL, D), jnp.float32),
        pltpu.VMEM((D8, DL, CL), jnp.float32),
        pltpu.VMEM((D8, DL, CL), jnp.float32),
        pltpu.SemaphoreType.DMA,
        pltpu.SemaphoreType.DMA,
        pltpu.SemaphoreType.DMA,
        pltpu.SemaphoreType.DMA,
    ],
    compiler_params=pltpu.CompilerParams(use_tc_tiling_on_sc=False,
                                         needs_layout_passes=False),
)
def _enc_kernel(w4_hbm, table_hbm, out_hbm,
                idxall, rows0, rows1, tt0, tt1, g0, g1, ws0, ws1):
    wrb = lax.axis_index("s") * NC + lax.axis_index("c")

    @pl.loop(0, SB)
    def _stage(sb):
        pltpu.sync_copy(w4_hbm.at[sb, wrb], idxall.at[sb])

    rows = (rows0, rows1)
    tt = (tt0, tt1)
    gsem = (g0, g1)
    wsem = (ws0, ws1)

    rl16 = lax.iota(jnp.int32, 16)
    rlv = [rl16 + 16 * k for k in range(8)]

    def start_gather(s, slot):
        sb = s // 8
        s8 = s - 8 * sb
        pltpu.async_copy(table_hbm.at[idxall.at[sb, s8]], rows[slot],
                         gsem[slot])

    def wait_gather(s, slot):
        sb = s // 8
        s8 = s - 8 * sb
        pltpu.make_async_copy(table_hbm.at[idxall.at[sb, s8]], rows[slot],
                              gsem[slot]).wait()

    def transpose_unit(slot):
        for d8 in range(D8):
            for dl in range(DL):
                d = d8 * 8 + dl
                dvec = jnp.full((16,), d, jnp.int32)
                for k in range(8):
                    val = plsc.load_gather(rows[slot], [rlv[k], dvec])
                    tt[slot][d8, dl, pl.ds(k * 16, 16)] = val

    def start_write(s, slot):
        pltpu.async_copy(tt[slot], out_hbm.at[s, :, wrb], wsem[slot])

    def drain_write(slot):
        pltpu.make_async_copy(out_hbm.at[0, :, wrb], tt[slot],
                              wsem[slot]).wait()

    start_gather(0, 0)

    @pl.loop(0, S, step=2)
    def _unit(s):
        start_gather(s + 1, 1)
        wait_gather(s, 0)

        @pl.when(s >= 2)
        def _():
            drain_write(0)
        transpose_unit(0)
        start_write(s, 0)

        @pl.when(s + 2 < S)
        def _():
            start_gather(s + 2, 0)
        wait_gather(s + 1, 1)

        @pl.when(s >= 2)
        def _():
            drain_write(1)
        transpose_unit(1)
        start_write(s + 1, 1)

    drain_write(0)
    drain_write(1)


def kernel(words, table):
    w4 = words.T.reshape(SB, S8, CB, CL).transpose(0, 2, 1, 3)
    out5 = _enc_kernel(w4, table)
    return out5.transpose(2, 4, 0, 1, 3).reshape(R, S, D)
